# baseline (speedup 1.0000x reference)
import functools

import jax
import jax.numpy as jnp
from jax import lax
from jax.experimental import pallas as pl
from jax.experimental.pallas import tpu as pltpu
from jax.experimental.pallas import tpu_sc as plsc

_BIGRAM_VOCAB = 1000000
_MOD = _BIGRAM_VOCAB - 1
_D = 64
_N = 1024
_B = 32768

_VB = 8192
_S = 499712
_L = 507904

_NC = 2
_NS = 16
_NW = _NC * _NS
_BPW = _B // _NW
_NSTREAM = 8
_IDX_W = _BPW // _NSTREAM


def _tr_body(xa_ref, xb_ref, o_ref):
    x = jnp.concatenate([xa_ref[...], xb_ref[...]], axis=0)
    o_ref[...] = x.T


_tr = pl.pallas_call(
    _tr_body,
    grid=(_L // _VB,),
    in_specs=[
        pl.BlockSpec((_D, _VB), lambda i: (0, i)),
        pl.BlockSpec((_D, _VB), lambda i: (0, i + _S // _VB)),
    ],
    out_specs=pl.BlockSpec((_VB, 2 * _D), lambda i: (i, 0)),
    out_shape=jax.ShapeDtypeStruct((_L, 2 * _D), jnp.float32),
)


_mesh = plsc.VectorSubcoreMesh(core_axis_name="c", subcore_axis_name="s")


@functools.partial(
    pl.kernel,
    out_type=(
        jax.ShapeDtypeStruct((_B, 2 * _D), jnp.float32),
        jax.ShapeDtypeStruct((_B,), jnp.float32),
    ),
    mesh=_mesh,
    scratch_types=[
        pltpu.VMEM((_BPW,), jnp.int32),
        pltpu.VMEM((_BPW,), jnp.int32),
        pltpu.VMEM((_NSTREAM, _IDX_W), jnp.int32),
        pltpu.VMEM((_BPW,), jnp.float32),
        pltpu.VMEM((_BPW // 2, 2 * _D), jnp.float32),
        pltpu.SemaphoreType.DMA,
    ],
)
def _sc_hash_gather(ta_hbm, tb_hbm, t2_hbm, g_hbm, sel_hbm, ta_v, tb_v,
                    idx_v, sel_v, rows_v, sem):
    wid = lax.axis_index("s") * _NC + lax.axis_index("c")
    base = wid * _BPW
    pltpu.sync_copy(ta_hbm.at[pl.ds(base, _BPW)], ta_v)
    pltpu.sync_copy(tb_hbm.at[pl.ds(base, _BPW)], tb_v)

    for j in range(_NSTREAM):
        @pl.loop(0, _IDX_W, step=16)
        def _(k, j=j):
            off = j * _IDX_W + k
            a = ta_v[pl.ds(off, 16)]
            b = tb_v[pl.ds(off, 16)]
            h = (jnp.int32(36313) * a) ^ (jnp.int32(27191) * b)
            r = lax.rem(h, jnp.int32(_MOD))
            r = jnp.where(r < 0, r + jnp.int32(_MOD), r)
            p = base + off + lax.iota(jnp.int32, 16)
            r = jnp.where(p == 0, jnp.int32(_MOD), r)
            hi = r >= jnp.int32(_S)
            idx_v[j, pl.ds(k, 16)] = jnp.where(hi, r - jnp.int32(_S), r)
            sel_v[pl.ds(off, 16)] = jnp.where(hi, jnp.float32(1.0),
                                              jnp.float32(0.0))

    for half in range(2):
        copies = [
            pltpu.async_copy(
                t2_hbm.at[idx_v.at[half * 4 + j]],
                rows_v.at[pl.ds(j * _IDX_W, _IDX_W)],
                sem,
            )
            for j in range(4)
        ]
        for c in copies:
            c.wait()
        pltpu.sync_copy(
            rows_v, g_hbm.at[pl.ds(base + half * (_BPW // 2), _BPW // 2)])
    pltpu.sync_copy(sel_v, sel_hbm.at[pl.ds(base, _BPW)])


_BM = 4096


def _mm_body(s_ref, x_ref, sel_ref, w_ref, o_ref):
    a = x_ref[:, :_D]
    b = x_ref[:, _D:]
    h = jnp.where(sel_ref[...] != 0, b, a)
    acc = jax.lax.dot_general(
        h, w_ref[...], (((1,), (0,)), ((), ())),
        preferred_element_type=jnp.float32,
    )
    o_ref[...] = acc * s_ref[0]


_mm = pl.pallas_call(
    _mm_body,
    grid=(_B // _BM,),
    in_specs=[
        pl.BlockSpec(memory_space=pltpu.SMEM),
        pl.BlockSpec((_BM, 2 * _D), lambda i: (i, 0)),
        pl.BlockSpec((_BM, 1), lambda i: (i, 0)),
        pl.BlockSpec((_D, _N), lambda i: (0, 0)),
    ],
    out_specs=pl.BlockSpec((_BM, _N), lambda i: (i, 0)),
    out_shape=jax.ShapeDtypeStruct((_B, _N), jnp.float32),
)


def kernel(token_ids, embed_table, proj_w, scale):
    tokens = token_ids.astype(jnp.int32)
    prev = jnp.roll(tokens, 1)
    tt = embed_table.T
    t2 = _tr(tt, tt)
    g2, sel = _sc_hash_gather(tokens, prev, t2)
    sel2d = sel.astype(jnp.int8).reshape(_B, 1)
    wt = proj_w.T
    s = jnp.reshape(scale.astype(jnp.float32), (1,))
    return _mm(s, g2, sel2d, wt)


# speedup vs baseline: 1.0179x; 1.0179x over previous
import functools

import jax
import jax.numpy as jnp
from jax import lax
from jax.experimental import pallas as pl
from jax.experimental.pallas import tpu as pltpu
from jax.experimental.pallas import tpu_sc as plsc

_BIGRAM_VOCAB = 1000000
_MOD = _BIGRAM_VOCAB - 1
_D = 64
_N = 1024
_B = 32768

_VB = 16384
_S = 507904
_L = 507904

_NC = 2
_NS = 16
_NW = _NC * _NS
_BPW = _B // _NW
_NSTREAM = 8
_IDX_W = _BPW // _NSTREAM


def _tr_body(xa_ref, xb_ref, o_ref):
    x = jnp.concatenate([xa_ref[...], xb_ref[...]], axis=0)
    o_ref[...] = x.T


_tr = pl.pallas_call(
    _tr_body,
    grid=(_L // _VB,),
    in_specs=[
        pl.BlockSpec((_D, _VB), lambda i: (0, i)),
        pl.BlockSpec((_D, _VB), lambda i: (0, i + _S // _VB)),
    ],
    out_specs=pl.BlockSpec((_VB, 2 * _D), lambda i: (i, 0)),
    out_shape=jax.ShapeDtypeStruct((_L, 2 * _D), jnp.float32),
)


_mesh = plsc.VectorSubcoreMesh(core_axis_name="c", subcore_axis_name="s")


@functools.partial(
    pl.kernel,
    out_type=(
        jax.ShapeDtypeStruct((_B, 2 * _D), jnp.float32),
        jax.ShapeDtypeStruct((_B,), jnp.float32),
    ),
    mesh=_mesh,
    scratch_types=[
        pltpu.VMEM((_BPW,), jnp.int32),
        pltpu.VMEM((_BPW,), jnp.int32),
        pltpu.VMEM((_NSTREAM, _IDX_W), jnp.int32),
        pltpu.VMEM((_BPW,), jnp.float32),
        pltpu.VMEM((_BPW // 2, 2 * _D), jnp.float32),
        pltpu.SemaphoreType.DMA,
    ],
)
def _sc_hash_gather(ta_hbm, tb_hbm, t2_hbm, g_hbm, sel_hbm, ta_v, tb_v,
                    idx_v, sel_v, rows_v, sem):
    wid = lax.axis_index("s") * _NC + lax.axis_index("c")
    base = wid * _BPW
    pltpu.sync_copy(ta_hbm.at[pl.ds(base, _BPW)], ta_v)
    pltpu.sync_copy(tb_hbm.at[pl.ds(base, _BPW)], tb_v)

    for j in range(_NSTREAM):
        @pl.loop(0, _IDX_W, step=16)
        def _(k, j=j):
            off = j * _IDX_W + k
            a = ta_v[pl.ds(off, 16)]
            b = tb_v[pl.ds(off, 16)]
            h = (jnp.int32(36313) * a) ^ (jnp.int32(27191) * b)
            r = lax.rem(h, jnp.int32(_MOD))
            r = jnp.where(r < 0, r + jnp.int32(_MOD), r)
            p = base + off + lax.iota(jnp.int32, 16)
            r = jnp.where(p == 0, jnp.int32(_MOD), r)
            hi = r >= jnp.int32(_S)
            idx_v[j, pl.ds(k, 16)] = jnp.where(hi, r - jnp.int32(_S), r)
            sel_v[pl.ds(off, 16)] = jnp.where(hi, jnp.float32(1.0),
                                              jnp.float32(0.0))

    for half in range(2):
        copies = [
            pltpu.async_copy(
                t2_hbm.at[idx_v.at[half * 4 + j]],
                rows_v.at[pl.ds(j * _IDX_W, _IDX_W)],
                sem,
            )
            for j in range(4)
        ]
        for c in copies:
            c.wait()
        pltpu.sync_copy(
            rows_v, g_hbm.at[pl.ds(base + half * (_BPW // 2), _BPW // 2)])
    pltpu.sync_copy(sel_v, sel_hbm.at[pl.ds(base, _BPW)])


_BM = 4096


def _mm_body(s_ref, x_ref, sel_ref, w_ref, o_ref):
    a = x_ref[:, :_D]
    b = x_ref[:, _D:]
    h = jnp.where(sel_ref[...] != 0, b, a)
    acc = jax.lax.dot_general(
        h, w_ref[...], (((1,), (0,)), ((), ())),
        preferred_element_type=jnp.float32,
    )
    o_ref[...] = acc * s_ref[0]


_mm = pl.pallas_call(
    _mm_body,
    grid=(_B // _BM,),
    in_specs=[
        pl.BlockSpec(memory_space=pltpu.SMEM),
        pl.BlockSpec((_BM, 2 * _D), lambda i: (i, 0)),
        pl.BlockSpec((_BM, 1), lambda i: (i, 0)),
        pl.BlockSpec((_D, _N), lambda i: (0, 0)),
    ],
    out_specs=pl.BlockSpec((_BM, _N), lambda i: (i, 0)),
    out_shape=jax.ShapeDtypeStruct((_B, _N), jnp.float32),
)


def kernel(token_ids, embed_table, proj_w, scale):
    tokens = token_ids.astype(jnp.int32)
    prev = jnp.roll(tokens, 1)
    tt = embed_table.T
    t2 = _tr(tt, tt)
    g2, sel = _sc_hash_gather(tokens, prev, t2)
    sel2d = sel.astype(jnp.int8).reshape(_B, 1)
    wt = proj_w.T
    s = jnp.reshape(scale.astype(jnp.float32), (1,))
    return _mm(s, g2, sel2d, wt)


# speedup vs baseline: 1.0230x; 1.0050x over previous
import functools

import jax
import jax.numpy as jnp
from jax import lax
from jax.experimental import pallas as pl
from jax.experimental.pallas import tpu as pltpu
from jax.experimental.pallas import tpu_sc as plsc

_BIGRAM_VOCAB = 1000000
_MOD = _BIGRAM_VOCAB - 1
_D = 64
_N = 1024
_B = 32768

_VB = 16384
_S = 507904
_L = 507904

_NC = 2
_NS = 16
_NW = _NC * _NS
_BPW = _B // _NW
_NSTREAM = 8
_IDX_W = _BPW // _NSTREAM


def _tr_body(xa_ref, xb_ref, o_ref):
    x = jnp.concatenate([xa_ref[...], xb_ref[...]], axis=0)
    o_ref[...] = x.T


_tr = pl.pallas_call(
    _tr_body,
    grid=(_L // _VB,),
    in_specs=[
        pl.BlockSpec((_D, _VB), lambda i: (0, i)),
        pl.BlockSpec((_D, _VB), lambda i: (0, i + _S // _VB)),
    ],
    out_specs=pl.BlockSpec((_VB, 2 * _D), lambda i: (i, 0)),
    out_shape=jax.ShapeDtypeStruct((_L, 2 * _D), jnp.float32),
    compiler_params=pltpu.CompilerParams(dimension_semantics=("parallel",)),
)


_mesh = plsc.VectorSubcoreMesh(core_axis_name="c", subcore_axis_name="s")


@functools.partial(
    pl.kernel,
    out_type=(
        jax.ShapeDtypeStruct((_B, 2 * _D), jnp.float32),
        jax.ShapeDtypeStruct((_B,), jnp.float32),
    ),
    mesh=_mesh,
    scratch_types=[
        pltpu.VMEM((_BPW,), jnp.int32),
        pltpu.VMEM((_BPW,), jnp.int32),
        pltpu.VMEM((_NSTREAM, _IDX_W), jnp.int32),
        pltpu.VMEM((_BPW,), jnp.float32),
        pltpu.VMEM((_BPW // 2, 2 * _D), jnp.float32),
        pltpu.SemaphoreType.DMA,
    ],
)
def _sc_hash_gather(ta_hbm, tb_hbm, t2_hbm, g_hbm, sel_hbm, ta_v, tb_v,
                    idx_v, sel_v, rows_v, sem):
    wid = lax.axis_index("s") * _NC + lax.axis_index("c")
    base = wid * _BPW
    pltpu.sync_copy(ta_hbm.at[pl.ds(base, _BPW)], ta_v)
    pltpu.sync_copy(tb_hbm.at[pl.ds(base, _BPW)], tb_v)

    for j in range(_NSTREAM):
        @pl.loop(0, _IDX_W, step=16)
        def _(k, j=j):
            off = j * _IDX_W + k
            a = ta_v[pl.ds(off, 16)]
            b = tb_v[pl.ds(off, 16)]
            h = (jnp.int32(36313) * a) ^ (jnp.int32(27191) * b)
            r = lax.rem(h, jnp.int32(_MOD))
            r = jnp.where(r < 0, r + jnp.int32(_MOD), r)
            p = base + off + lax.iota(jnp.int32, 16)
            r = jnp.where(p == 0, jnp.int32(_MOD), r)
            hi = r >= jnp.int32(_S)
            idx_v[j, pl.ds(k, 16)] = jnp.where(hi, r - jnp.int32(_S), r)
            sel_v[pl.ds(off, 16)] = jnp.where(hi, jnp.float32(1.0),
                                              jnp.float32(0.0))

    for half in range(2):
        copies = [
            pltpu.async_copy(
                t2_hbm.at[idx_v.at[half * 4 + j]],
                rows_v.at[pl.ds(j * _IDX_W, _IDX_W)],
                sem,
            )
            for j in range(4)
        ]
        for c in copies:
            c.wait()
        pltpu.sync_copy(
            rows_v, g_hbm.at[pl.ds(base + half * (_BPW // 2), _BPW // 2)])
    pltpu.sync_copy(sel_v, sel_hbm.at[pl.ds(base, _BPW)])


_BM = 4096


def _mm_body(s_ref, x_ref, sel_ref, w_ref, o_ref):
    a = x_ref[:, :_D]
    b = x_ref[:, _D:]
    h = jnp.where(sel_ref[...] != 0, b, a)
    acc = jax.lax.dot_general(
        h, w_ref[...], (((1,), (0,)), ((), ())),
        preferred_element_type=jnp.float32,
    )
    o_ref[...] = acc * s_ref[0]


_mm = pl.pallas_call(
    _mm_body,
    grid=(_B // _BM,),
    in_specs=[
        pl.BlockSpec(memory_space=pltpu.SMEM),
        pl.BlockSpec((_BM, 2 * _D), lambda i: (i, 0)),
        pl.BlockSpec((_BM, 1), lambda i: (i, 0)),
        pl.BlockSpec((_D, _N), lambda i: (0, 0)),
    ],
    out_specs=pl.BlockSpec((_BM, _N), lambda i: (i, 0)),
    out_shape=jax.ShapeDtypeStruct((_B, _N), jnp.float32),
    compiler_params=pltpu.CompilerParams(dimension_semantics=("parallel",)),
)


def kernel(token_ids, embed_table, proj_w, scale):
    tokens = token_ids.astype(jnp.int32)
    prev = jnp.roll(tokens, 1)
    tt = embed_table.T
    t2 = _tr(tt, tt)
    g2, sel = _sc_hash_gather(tokens, prev, t2)
    sel2d = sel.astype(jnp.int8).reshape(_B, 1)
    wt = proj_w.T
    s = jnp.reshape(scale.astype(jnp.float32), (1,))
    return _mm(s, g2, sel2d, wt)
